# baseline (speedup 1.0000x reference)
import functools

import jax
import jax.numpy as jnp
from jax import lax
from jax.experimental import pallas as pl
from jax.experimental.pallas import tpu as pltpu
from jax.experimental.pallas import tpu_sc as plsc

_CAPACITY = 0.125
_INT_MIN = -2147483648
_NC = 2
_NS = 16
_L = 16


def _select(rw, k):
    bits = lax.bitcast_convert_type(rw, jnp.int32)
    key = jnp.where(bits < 0, bits ^ 0x7FFFFFFF, bits)
    B, S = rw.shape

    def t_step(i, t):
        bit = 31 - i
        cand = t | (1 << bit)
        cand_s = cand ^ _INT_MIN
        cnt = jnp.sum((key >= cand_s).astype(jnp.int32), axis=1, keepdims=True)
        return jnp.where(cnt >= k, cand, t)

    t = lax.fori_loop(0, 32, t_step, jnp.zeros((B, 1), jnp.int32))
    thr = t ^ _INT_MIN

    gt = key > thr
    c_gt = jnp.sum(gt.astype(jnp.int32), axis=1, keepdims=True)
    need = k - c_gt
    eq = key == thr
    idx = lax.broadcasted_iota(jnp.int32, (B, S), 1)

    def c_step(i, t2):
        bit = 13 - i
        cand = t2 | (1 << bit)
        f = jnp.sum((eq & (idx < cand)).astype(jnp.int32), axis=1, keepdims=True)
        return jnp.where(f < need, cand, t2)

    t2 = lax.fori_loop(0, 14, c_step, jnp.zeros((B, 1), jnp.int32))
    return gt | (eq & (idx <= t2))


def _prep_body(x_ref, wr_ref, o_ref, m_ref, rw_ref, rw_s, *, n, k, rb):
    i = pl.program_id(0)

    @pl.when(i < n)
    def _copy_and_route():
        xb = x_ref[...]
        o_ref[...] = xb
        xb16 = xb.astype(jnp.bfloat16).astype(jnp.float32)
        wr16 = wr_ref[...].astype(jnp.bfloat16).astype(jnp.float32)
        rwb = jnp.sum(xb16 * wr16, axis=1)
        b = i // 8
        col = (i % 8) * rb
        rw_s[pl.ds(b, 1), pl.ds(col, rb)] = rwb.reshape(1, rb)

    @pl.when(i == n)
    def _select_step():
        rw = rw_s[...]
        rw_ref[...] = rw
        m_ref[...] = _select(rw, k).astype(jnp.float32)


def _gat(v, idx):
    return v.at[idx].get(mode="promise_in_bounds")


def _compact_body(mask_hbm, rw_hbm, topi, selw, mvm, rvm, tvm, wvm, *, S, k):
    c = lax.axis_index("c")
    s = lax.axis_index("s")
    wid = s * _NC + c

    @pl.when(wid < 4)
    def _():
        b = wid
        pltpu.sync_copy(mask_hbm.at[b], mvm)
        pltpu.sync_copy(rw_hbm.at[b], rvm)
        lanes = lax.broadcasted_iota(jnp.int32, (_L,), 0)
        zero = lanes * 0

        def body(ci, off):
            m = mvm[pl.ds(ci * _L, _L)] != 0.0
            w = rvm[pl.ds(ci * _L, _L)]
            gi = b * S + ci * _L + lanes
            p = jnp.where(m, zero + 1, zero)
            for sh in (1, 2, 4, 8):
                g = _gat(p, jnp.where(lanes >= sh, lanes - sh, zero))
                p = p + jnp.where(lanes >= sh, g, zero)
            tgt = lanes + 1
            iv = zero
            for sh in (8, 4, 2, 1):
                cand = iv + sh
                pv = _gat(p, cand - 1)
                iv = jnp.where(pv < tgt, cand, iv)
            ivc = jnp.where(iv > 15, zero + 15, iv)
            tvm[pl.ds(off, _L)] = _gat(gi, ivc)
            wvm[pl.ds(off, _L)] = _gat(w, ivc)
            return off + p[15]

        lax.fori_loop(0, S // _L, body, b * 0)
        pltpu.sync_copy(tvm.at[pl.ds(0, k)], topi.at[pl.ds(b * k, k)])
        pltpu.sync_copy(wvm.at[pl.ds(0, k)], selw.at[pl.ds(b * k, k)])


def _gather_body(topi, x2, sel, idxv, rows, sem, *, rpw):
    c = lax.axis_index("c")
    s = lax.axis_index("s")
    base = (s * _NC + c) * rpw
    pltpu.sync_copy(topi.at[pl.ds(base, rpw)], idxv)
    pltpu.async_copy(x2.at[idxv], rows, sem).wait()
    pltpu.sync_copy(rows, sel.at[pl.ds(base, rpw)])


def _scatter_body(out_hbm, topi, trans, idxv, rows, sem, *, rpw):
    c = lax.axis_index("c")
    s = lax.axis_index("s")
    base = (s * _NC + c) * rpw
    pltpu.sync_copy(topi.at[pl.ds(base, rpw)], idxv)
    pltpu.sync_copy(trans.at[pl.ds(base, rpw)], rows)
    pltpu.async_copy(rows, out_hbm.at[idxv], sem).wait()


def _matmul_body(sel_ref, w_ref, wt_ref, bt_ref, br_ref, o_ref):
    sb = sel_ref[...]
    y = jnp.dot(sb.astype(jnp.bfloat16), wt_ref[...].astype(jnp.bfloat16),
                preferred_element_type=jnp.float32)
    y = y + bt_ref[...]
    w = w_ref[0, 0, :] + br_ref[0, 0]
    o_ref[...] = y * w[:, None]


def kernel(x, Wr, br, Wa, ba, Wt, bt):
    B, S, D = x.shape
    k = int(S * _CAPACITY)
    RB = 1024
    n = (B * S) // RB
    x2 = x.reshape(B * S, D)
    nsel = B * k
    rpw = nsel // (_NC * _NS)

    out2, mask, rw = pl.pallas_call(
        functools.partial(_prep_body, n=n, k=k, rb=RB),
        grid=(n + 1,),
        in_specs=[
            pl.BlockSpec((RB, D), lambda i: (jnp.minimum(i, 31), 0)),
            pl.BlockSpec((1, D), lambda i: (0, 0)),
        ],
        out_specs=(
            pl.BlockSpec((RB, D), lambda i: (jnp.minimum(i, 31), 0)),
            pl.BlockSpec((B, S), lambda i: (0, 0)),
            pl.BlockSpec((B, S), lambda i: (0, 0)),
        ),
        out_shape=(
            jax.ShapeDtypeStruct((B * S, D), jnp.float32),
            jax.ShapeDtypeStruct((B, S), jnp.float32),
            jax.ShapeDtypeStruct((B, S), jnp.float32),
        ),
        scratch_shapes=[pltpu.VMEM((B, S), jnp.float32)],
    )(x2, Wr.reshape(1, D))

    mesh = plsc.VectorSubcoreMesh(core_axis_name="c", subcore_axis_name="s")

    topi, selw = pl.kernel(
        functools.partial(_compact_body, S=S, k=k),
        out_type=(
            jax.ShapeDtypeStruct((nsel,), jnp.int32),
            jax.ShapeDtypeStruct((nsel,), jnp.float32),
        ),
        mesh=mesh,
        scratch_types=[
            pltpu.VMEM((S,), jnp.float32),
            pltpu.VMEM((S,), jnp.float32),
            pltpu.VMEM((k + _L,), jnp.int32),
            pltpu.VMEM((k + _L,), jnp.float32),
        ],
    )(mask, rw)

    sel = pl.kernel(
        functools.partial(_gather_body, rpw=rpw),
        out_type=jax.ShapeDtypeStruct((nsel, D), jnp.float32),
        mesh=mesh,
        scratch_types=[
            pltpu.VMEM((rpw,), jnp.int32),
            pltpu.VMEM((rpw, D), jnp.float32),
            pltpu.SemaphoreType.DMA,
        ],
    )(topi, x2)

    trans = pl.pallas_call(
        _matmul_body,
        grid=(B,),
        in_specs=[
            pl.BlockSpec((k, D), lambda i: (i, 0)),
            pl.BlockSpec((1, 1, k), lambda i: (i, 0, 0)),
            pl.BlockSpec((D, D), lambda i: (0, 0)),
            pl.BlockSpec((1, D), lambda i: (0, 0)),
            pl.BlockSpec((1, 1), lambda i: (0, 0)),
        ],
        out_specs=pl.BlockSpec((k, D), lambda i: (i, 0)),
        out_shape=jax.ShapeDtypeStruct((nsel, D), jnp.float32),
    )(sel, selw.reshape(B, 1, k), Wt, bt.reshape(1, D), br.reshape(1, 1))

    acc = jax.new_ref(out2)
    pl.kernel(
        functools.partial(_scatter_body, rpw=rpw),
        out_type=(),
        mesh=mesh,
        scratch_types=[
            pltpu.VMEM((rpw,), jnp.int32),
            pltpu.VMEM((rpw, D), jnp.float32),
            pltpu.SemaphoreType.DMA,
        ],
    )(acc, topi, trans)

    return acc[...].reshape(B, S, D)
